# baseline (speedup 1.0000x reference)
import functools

import jax
import jax.numpy as jnp
from jax import lax
from jax.experimental import pallas as pl
from jax.experimental.pallas import tpu as pltpu
from jax.experimental.pallas import tpu_sc as plsc

N_NODES = 10000
N_EDGES = 320000
D_FEAT = 128
LO, HI = 0.6, 0.8

NC, NS, L = 2, 16, 16
NW = NC * NS
E_W = N_EDGES // NW
G = 80
NG = E_W // G

_LN2 = 0.6931471805599453
_LN_COEFFS = (
    -0.0023178547278233695, 0.0037526242174348444, -0.004702811759193969,
    0.00813297750899996, -0.014655236243441047, 0.02636311409790927,
    -0.0493813242144068, 0.09876426659733828, -0.22222225241828053,
    0.6666666814031865, 0.4054651082139449,
)


def _ln(x):
    bits = plsc.bitcast(x, jnp.int32)
    e = lax.shift_right_logical(bits, 23) - 127
    m = plsc.bitcast((bits & 0x007FFFFF) | 0x3F800000, jnp.float32)
    w = m - 1.5
    r = jnp.full((L,), _LN_COEFFS[0], jnp.float32)
    for c in _LN_COEFFS[1:]:
        r = r * w + c
    return e.astype(jnp.float32) * _LN2 + r


def _rsqrt(s):
    r = plsc.bitcast(0x5F3759DF - lax.shift_right_logical(plsc.bitcast(s, jnp.int32), 1),
                     jnp.float32)
    for _ in range(3):
        r = r * (1.5 - 0.5 * s * r * r)
    return r


_mesh = plsc.VectorSubcoreMesh(core_axis_name="c", subcore_axis_name="s",
                               num_cores=NC, num_subcores=NS)


@functools.partial(
    pl.kernel,
    out_type=jax.ShapeDtypeStruct((NW, 4 * L), jnp.float32),
    mesh=_mesh,
    compiler_params=pltpu.CompilerParams(needs_layout_passes=False),
    scratch_types=[
        pltpu.VMEM((E_W,), jnp.int32),
        pltpu.VMEM((E_W,), jnp.int32),
        pltpu.VMEM((N_NODES,), jnp.float32),
        pltpu.VMEM((G, D_FEAT), jnp.float32),
        pltpu.VMEM((G, D_FEAT), jnp.float32),
        pltpu.VMEM((G, D_FEAT), jnp.float32),
        pltpu.VMEM((G, D_FEAT), jnp.float32),
        pltpu.VMEM((4 * L,), jnp.float32),
        pltpu.SemaphoreType.DMA,
        pltpu.SemaphoreType.DMA,
        pltpu.SemaphoreType.DMA,
        pltpu.SemaphoreType.DMA,
    ],
)
def _partials(e0_hbm, e1_hbm, probas_hbm, feats_hbm, out_hbm,
              e0_v, e1_v, probas_v, r0a, r1a, r0b, r1b, out_v,
              s0a, s1a, s0b, s1b):
    lanes = lax.iota(jnp.int32, L)
    wid = lax.axis_index("s") * NC + lax.axis_index("c")
    base = wid * E_W
    pltpu.sync_copy(probas_hbm, probas_v)
    pltpu.sync_copy(e0_hbm.at[pl.ds(base, E_W)], e0_v)
    pltpu.sync_copy(e1_hbm.at[pl.ds(base, E_W)], e1_v)

    def issue(g, r0, r1, s0, s1):
        off = g * G
        pltpu.async_copy(feats_hbm.at[e0_v.at[pl.ds(off, G)]], r0, s0)
        pltpu.async_copy(feats_hbm.at[e1_v.at[pl.ds(off, G)]], r1, s1)

    def wait(r0, r1, s0, s1):
        pltpu.make_async_copy(feats_hbm.at[e0_v.at[pl.ds(0, G)]], r0, s0).wait()
        pltpu.make_async_copy(feats_hbm.at[e1_v.at[pl.ds(0, G)]], r1, s1).wait()

    def compute(g, r0, r1, carry):
        off = g * G

        def sub_body(sub, carry):
            sp, sn, cp, cn = carry
            base_row = sub * L
            ssq = jnp.zeros((L,), jnp.float32)
            for eb in range(0, L, 4):
                accs = [None] * 4
                for j in range(D_FEAT // L):
                    for k in range(4):
                        row = base_row + eb + k
                        dv = (r0[row, pl.ds(j * L, L)]
                              - r1[row, pl.ds(j * L, L)])
                        sq = dv * dv
                        accs[k] = sq if accs[k] is None else accs[k] + sq
                for k in range(4):
                    ssq = jnp.where(lanes == eb + k, jnp.sum(accs[k]), ssq)
            i0 = e0_v[pl.ds(off + sub * L, L)]
            i1 = e1_v[pl.ds(off + sub * L, L)]
            p0 = plsc.load_gather(probas_v, [i0])
            p1 = plsc.load_gather(probas_v, [i1])
            hi0, hi1 = p0 >= HI, p1 >= HI
            lo0, lo1 = p0 < LO, p1 < LO
            sim = (hi0 & hi1) | (lo0 & lo1)
            dis = (hi0 & lo1) | (hi1 & lo0)
            norm = jnp.where(ssq > 1e-37, ssq * _rsqrt(ssq), 0.0)
            u = jnp.exp(-norm)
            t = 1.0 - u
            lnt = _ln(jnp.maximum(t, 1e-30))
            neg_t = jnp.where(t > 0.0, jnp.minimum(-lnt, 100.0), 100.0)
            sp = sp + jnp.where(sim, jnp.minimum(norm, 100.0), 0.0)
            sn = sn + jnp.where(dis, neg_t, 0.0)
            cp = cp + jnp.where(sim, 1.0, 0.0)
            cn = cn + jnp.where(dis, 1.0, 0.0)
            return sp, sn, cp, cn

        return lax.fori_loop(0, G // L, sub_body, carry)

    issue(0, r0a, r1a, s0a, s1a)
    z = jnp.zeros((L,), jnp.float32)

    def block2(k, carry):
        g0 = 2 * k
        issue(g0 + 1, r0b, r1b, s0b, s1b)
        wait(r0a, r1a, s0a, s1a)
        carry = compute(g0, r0a, r1a, carry)
        issue(g0 + 2, r0a, r1a, s0a, s1a)
        wait(r0b, r1b, s0b, s1b)
        return compute(g0 + 1, r0b, r1b, carry)

    carry = lax.fori_loop(0, NG // 2, block2, (z, z, z, z))
    wait(r0a, r1a, s0a, s1a)
    sp, sn, cp, cn = compute(NG - 1, r0a, r1a, carry)
    out_v[pl.ds(0, L)] = sp
    out_v[pl.ds(L, L)] = sn
    out_v[pl.ds(2 * L, L)] = cp
    out_v[pl.ds(3 * L, L)] = cn
    pltpu.sync_copy(out_v, out_hbm.at[wid])


def kernel(edges_nn, probas, feats):
    e0 = edges_nn[:, 0].astype(jnp.int32)
    e1 = edges_nn[:, 1].astype(jnp.int32)
    parts = _partials(e0, e1, probas, feats)
    q = parts.reshape(NW, 4, L).sum(axis=(0, 2))
    s_pos, s_neg, n_pos, n_neg = q[0], q[1], q[2], q[3]
    n_max = jnp.maximum(n_pos, n_neg)
    pos_w = jnp.where(n_pos > 0, n_max / n_pos, 0.0)
    neg_w = jnp.where(n_neg > 0, n_max / n_neg, 0.0)
    return (pos_w * s_pos + neg_w * s_neg) / (n_pos + n_neg)


# speedup vs baseline: 1.5138x; 1.5138x over previous
import functools

import jax
import jax.numpy as jnp
from jax import lax
from jax.experimental import pallas as pl
from jax.experimental.pallas import tpu as pltpu
from jax.experimental.pallas import tpu_sc as plsc

N_NODES = 10000
N_EDGES = 320000
D_FEAT = 128
LO, HI = 0.6, 0.8

NC, NS, L = 2, 16, 16
NW = NC * NS
E_W = N_EDGES // NW
G = 80
NG = E_W // G

_LN2 = 0.6931471805599453
_LN_COEFFS = (
    -0.0023178547278233695, 0.0037526242174348444, -0.004702811759193969,
    0.00813297750899996, -0.014655236243441047, 0.02636311409790927,
    -0.0493813242144068, 0.09876426659733828, -0.22222225241828053,
    0.6666666814031865, 0.4054651082139449,
)


def _ln(x):
    bits = plsc.bitcast(x, jnp.int32)
    e = lax.shift_right_logical(bits, 23) - 127
    m = plsc.bitcast((bits & 0x007FFFFF) | 0x3F800000, jnp.float32)
    w = m - 1.5
    r = jnp.full((L,), _LN_COEFFS[0], jnp.float32)
    for c in _LN_COEFFS[1:]:
        r = r * w + c
    return e.astype(jnp.float32) * _LN2 + r


def _rsqrt(s):
    r = plsc.bitcast(0x5F3759DF - lax.shift_right_logical(plsc.bitcast(s, jnp.int32), 1),
                     jnp.float32)
    for _ in range(3):
        r = r * (1.5 - 0.5 * s * r * r)
    return r


_mesh = plsc.VectorSubcoreMesh(core_axis_name="c", subcore_axis_name="s",
                               num_cores=NC, num_subcores=NS)


@functools.partial(
    pl.kernel,
    out_type=jax.ShapeDtypeStruct((NW, 4 * L), jnp.float32),
    mesh=_mesh,
    compiler_params=pltpu.CompilerParams(needs_layout_passes=False),
    scratch_types=[
        pltpu.VMEM((E_W,), jnp.int32),
        pltpu.VMEM((E_W,), jnp.int32),
        pltpu.VMEM((N_NODES,), jnp.float32),
        pltpu.VMEM((G, D_FEAT), jnp.float32),
        pltpu.VMEM((G, D_FEAT), jnp.float32),
        pltpu.VMEM((G, D_FEAT), jnp.float32),
        pltpu.VMEM((G, D_FEAT), jnp.float32),
        pltpu.VMEM((L,), jnp.float32),
        pltpu.VMEM((4 * L,), jnp.float32),
        pltpu.SemaphoreType.DMA,
        pltpu.SemaphoreType.DMA,
        pltpu.SemaphoreType.DMA,
        pltpu.SemaphoreType.DMA,
    ],
)
def _partials(e0_hbm, e1_hbm, probas_hbm, feats_hbm, out_hbm,
              e0_v, e1_v, probas_v, r0a, r1a, r0b, r1b, ssqbuf_v, out_v,
              s0a, s1a, s0b, s1b):
    lanes = lax.iota(jnp.int32, L)
    wid = lax.axis_index("s") * NC + lax.axis_index("c")
    base = wid * E_W
    pltpu.sync_copy(probas_hbm, probas_v)
    pltpu.sync_copy(e0_hbm.at[pl.ds(base, E_W)], e0_v)
    pltpu.sync_copy(e1_hbm.at[pl.ds(base, E_W)], e1_v)

    def issue(g, r0, r1, s0, s1):
        off = g * G
        pltpu.async_copy(feats_hbm.at[e0_v.at[pl.ds(off, G)]], r0, s0)
        pltpu.async_copy(feats_hbm.at[e1_v.at[pl.ds(off, G)]], r1, s1)

    def wait(r0, r1, s0, s1):
        pltpu.make_async_copy(feats_hbm.at[e0_v.at[pl.ds(0, G)]], r0, s0).wait()
        pltpu.make_async_copy(feats_hbm.at[e1_v.at[pl.ds(0, G)]], r1, s1).wait()

    def compute(g, r0, r1, carry):
        off = g * G

        def sub_body(sub, carry):
            sp, sn, cp, cn = carry
            base_row = sub * L
            ssqbuf_v[...] = jnp.zeros((L,), jnp.float32)
            for eb in range(0, L, 4):
                accs = [None] * 4
                for j in range(D_FEAT // L):
                    for k in range(4):
                        row = base_row + eb + k
                        dv = (r0[row, pl.ds(j * L, L)]
                              - r1[row, pl.ds(j * L, L)])
                        sq = dv * dv
                        accs[k] = sq if accs[k] is None else accs[k] + sq
                for k in range(4):
                    plsc.addupdate_scatter(
                        ssqbuf_v, [jnp.full((L,), eb + k, jnp.int32)], accs[k])
            ssq = ssqbuf_v[...]
            i0 = e0_v[pl.ds(off + sub * L, L)]
            i1 = e1_v[pl.ds(off + sub * L, L)]
            p0 = plsc.load_gather(probas_v, [i0])
            p1 = plsc.load_gather(probas_v, [i1])
            hi0, hi1 = p0 >= HI, p1 >= HI
            lo0, lo1 = p0 < LO, p1 < LO
            sim = (hi0 & hi1) | (lo0 & lo1)
            dis = (hi0 & lo1) | (hi1 & lo0)
            norm = jnp.where(ssq > 1e-37, ssq * _rsqrt(ssq), 0.0)
            u = jnp.exp(-norm)
            t = 1.0 - u
            lnt = _ln(jnp.maximum(t, 1e-30))
            neg_t = jnp.where(t > 0.0, jnp.minimum(-lnt, 100.0), 100.0)
            sp = sp + jnp.where(sim, jnp.minimum(norm, 100.0), 0.0)
            sn = sn + jnp.where(dis, neg_t, 0.0)
            cp = cp + jnp.where(sim, 1.0, 0.0)
            cn = cn + jnp.where(dis, 1.0, 0.0)
            return sp, sn, cp, cn

        return lax.fori_loop(0, G // L, sub_body, carry)

    issue(0, r0a, r1a, s0a, s1a)
    z = jnp.zeros((L,), jnp.float32)

    def block2(k, carry):
        g0 = 2 * k
        issue(g0 + 1, r0b, r1b, s0b, s1b)
        wait(r0a, r1a, s0a, s1a)
        carry = compute(g0, r0a, r1a, carry)
        issue(g0 + 2, r0a, r1a, s0a, s1a)
        wait(r0b, r1b, s0b, s1b)
        return compute(g0 + 1, r0b, r1b, carry)

    carry = lax.fori_loop(0, NG // 2, block2, (z, z, z, z))
    wait(r0a, r1a, s0a, s1a)
    sp, sn, cp, cn = compute(NG - 1, r0a, r1a, carry)
    out_v[pl.ds(0, L)] = sp
    out_v[pl.ds(L, L)] = sn
    out_v[pl.ds(2 * L, L)] = cp
    out_v[pl.ds(3 * L, L)] = cn
    pltpu.sync_copy(out_v, out_hbm.at[wid])


def kernel(edges_nn, probas, feats):
    e0 = edges_nn[:, 0].astype(jnp.int32)
    e1 = edges_nn[:, 1].astype(jnp.int32)
    parts = _partials(e0, e1, probas, feats)
    q = parts.reshape(NW, 4, L).sum(axis=(0, 2))
    s_pos, s_neg, n_pos, n_neg = q[0], q[1], q[2], q[3]
    n_max = jnp.maximum(n_pos, n_neg)
    pos_w = jnp.where(n_pos > 0, n_max / n_pos, 0.0)
    neg_w = jnp.where(n_neg > 0, n_max / n_neg, 0.0)
    return (pos_w * s_pos + neg_w * s_neg) / (n_pos + n_neg)


# speedup vs baseline: 1.6229x; 1.0720x over previous
import functools

import jax
import jax.numpy as jnp
from jax import lax
from jax.experimental import pallas as pl
from jax.experimental.pallas import tpu as pltpu
from jax.experimental.pallas import tpu_sc as plsc

N_NODES = 10000
N_EDGES = 320000
D_FEAT = 128
LO, HI = 0.6, 0.8

NC, NS, L = 2, 16, 16
NW = NC * NS
E_W = N_EDGES // NW
G = 80
NG = E_W // G

_LN2 = 0.6931471805599453
_LN_COEFFS = (
    -0.0023178547278233695, 0.0037526242174348444, -0.004702811759193969,
    0.00813297750899996, -0.014655236243441047, 0.02636311409790927,
    -0.0493813242144068, 0.09876426659733828, -0.22222225241828053,
    0.6666666814031865, 0.4054651082139449,
)


def _ln(x):
    bits = plsc.bitcast(x, jnp.int32)
    e = lax.shift_right_logical(bits, 23) - 127
    m = plsc.bitcast((bits & 0x007FFFFF) | 0x3F800000, jnp.float32)
    w = m - 1.5
    r = jnp.full((L,), _LN_COEFFS[0], jnp.float32)
    for c in _LN_COEFFS[1:]:
        r = r * w + c
    return e.astype(jnp.float32) * _LN2 + r


def _rsqrt(s):
    r = plsc.bitcast(0x5F3759DF - lax.shift_right_logical(plsc.bitcast(s, jnp.int32), 1),
                     jnp.float32)
    for _ in range(3):
        r = r * (1.5 - 0.5 * s * r * r)
    return r


_mesh = plsc.VectorSubcoreMesh(core_axis_name="c", subcore_axis_name="s",
                               num_cores=NC, num_subcores=NS)


@functools.partial(
    pl.kernel,
    out_type=jax.ShapeDtypeStruct((NW, 4 * L), jnp.float32),
    mesh=_mesh,
    compiler_params=pltpu.CompilerParams(needs_layout_passes=False,
                                         use_tc_tiling_on_sc=False),
    scratch_types=[
        pltpu.VMEM((E_W,), jnp.int32),
        pltpu.VMEM((E_W,), jnp.int32),
        pltpu.VMEM((N_NODES,), jnp.float32),
        pltpu.VMEM((G, D_FEAT), jnp.bfloat16),
        pltpu.VMEM((G, D_FEAT), jnp.bfloat16),
        pltpu.VMEM((G, D_FEAT), jnp.bfloat16),
        pltpu.VMEM((G, D_FEAT), jnp.bfloat16),
        pltpu.VMEM((L,), jnp.float32),
        pltpu.VMEM((4 * L,), jnp.float32),
        pltpu.SemaphoreType.DMA,
        pltpu.SemaphoreType.DMA,
        pltpu.SemaphoreType.DMA,
        pltpu.SemaphoreType.DMA,
    ],
)
def _partials(e0_hbm, e1_hbm, probas_hbm, feats_hbm, out_hbm,
              e0_v, e1_v, probas_v, r0a, r1a, r0b, r1b, ssqbuf_v, out_v,
              s0a, s1a, s0b, s1b):
    lanes = lax.iota(jnp.int32, L)
    wid = lax.axis_index("s") * NC + lax.axis_index("c")
    base = wid * E_W
    pltpu.sync_copy(probas_hbm, probas_v)
    pltpu.sync_copy(e0_hbm.at[pl.ds(base, E_W)], e0_v)
    pltpu.sync_copy(e1_hbm.at[pl.ds(base, E_W)], e1_v)

    def issue(g, r0, r1, s0, s1):
        off = g * G
        pltpu.async_copy(feats_hbm.at[e0_v.at[pl.ds(off, G)]], r0, s0)
        pltpu.async_copy(feats_hbm.at[e1_v.at[pl.ds(off, G)]], r1, s1)

    def wait(r0, r1, s0, s1):
        pltpu.make_async_copy(feats_hbm.at[e0_v.at[pl.ds(0, G)]], r0, s0).wait()
        pltpu.make_async_copy(feats_hbm.at[e1_v.at[pl.ds(0, G)]], r1, s1).wait()

    def compute(g, r0, r1, carry):
        off = g * G

        def sub_body(sub, carry):
            sp, sn, cp, cn = carry
            base_row = sub * L
            ssqbuf_v[...] = jnp.zeros((L,), jnp.float32)
            for eb in range(0, L, 4):
                accs = [None] * 4
                for j in range(D_FEAT // (2 * L)):
                    for k in range(4):
                        row = base_row + eb + k
                        dv = (r0[row, pl.ds(j * 2 * L, 2 * L)]
                              - r1[row, pl.ds(j * 2 * L, 2 * L)])
                        sq = dv * dv
                        sq_a, sq_b = plsc.unpack(
                            sq, format=plsc.PackFormat.INTERLEAVED)
                        part = sq_a + sq_b
                        accs[k] = part if accs[k] is None else accs[k] + part
                for k in range(4):
                    plsc.addupdate_scatter(
                        ssqbuf_v, [jnp.full((L,), eb + k, jnp.int32)], accs[k])
            ssq = ssqbuf_v[...]
            i0 = e0_v[pl.ds(off + sub * L, L)]
            i1 = e1_v[pl.ds(off + sub * L, L)]
            p0 = plsc.load_gather(probas_v, [i0])
            p1 = plsc.load_gather(probas_v, [i1])
            hi0, hi1 = p0 >= HI, p1 >= HI
            lo0, lo1 = p0 < LO, p1 < LO
            sim = (hi0 & hi1) | (lo0 & lo1)
            dis = (hi0 & lo1) | (hi1 & lo0)
            norm = jnp.where(ssq > 1e-37, ssq * _rsqrt(ssq), 0.0)
            u = jnp.exp(-norm)
            t = 1.0 - u
            lnt = _ln(jnp.maximum(t, 1e-30))
            neg_t = jnp.where(t > 0.0, jnp.minimum(-lnt, 100.0), 100.0)
            sp = sp + jnp.where(sim, jnp.minimum(norm, 100.0), 0.0)
            sn = sn + jnp.where(dis, neg_t, 0.0)
            cp = cp + jnp.where(sim, 1.0, 0.0)
            cn = cn + jnp.where(dis, 1.0, 0.0)
            return sp, sn, cp, cn

        return lax.fori_loop(0, G // L, sub_body, carry)

    issue(0, r0a, r1a, s0a, s1a)
    z = jnp.zeros((L,), jnp.float32)

    def block2(k, carry):
        g0 = 2 * k
        issue(g0 + 1, r0b, r1b, s0b, s1b)
        wait(r0a, r1a, s0a, s1a)
        carry = compute(g0, r0a, r1a, carry)
        issue(g0 + 2, r0a, r1a, s0a, s1a)
        wait(r0b, r1b, s0b, s1b)
        return compute(g0 + 1, r0b, r1b, carry)

    carry = lax.fori_loop(0, NG // 2, block2, (z, z, z, z))
    wait(r0a, r1a, s0a, s1a)
    sp, sn, cp, cn = compute(NG - 1, r0a, r1a, carry)
    out_v[pl.ds(0, L)] = sp
    out_v[pl.ds(L, L)] = sn
    out_v[pl.ds(2 * L, L)] = cp
    out_v[pl.ds(3 * L, L)] = cn
    pltpu.sync_copy(out_v, out_hbm.at[wid])


def kernel(edges_nn, probas, feats):
    e0 = edges_nn[:, 0].astype(jnp.int32)
    e1 = edges_nn[:, 1].astype(jnp.int32)
    parts = _partials(e0, e1, probas, feats.astype(jnp.bfloat16))
    q = parts.reshape(NW, 4, L).sum(axis=(0, 2))
    s_pos, s_neg, n_pos, n_neg = q[0], q[1], q[2], q[3]
    n_max = jnp.maximum(n_pos, n_neg)
    pos_w = jnp.where(n_pos > 0, n_max / n_pos, 0.0)
    neg_w = jnp.where(n_neg > 0, n_max / n_neg, 0.0)
    return (pos_w * s_pos + neg_w * s_neg) / (n_pos + n_neg)


# speedup vs baseline: 2.4184x; 1.4902x over previous
import functools

import jax
import jax.numpy as jnp
from jax import lax
from jax.experimental import pallas as pl
from jax.experimental.pallas import tpu as pltpu
from jax.experimental.pallas import tpu_sc as plsc

N_NODES = 10000
N_EDGES = 320000
D_FEAT = 128
LO, HI = 0.6, 0.8

NC, NS, L = 2, 16, 16
NW = NC * NS
E_W = N_EDGES // NW
G = 80
NG = E_W // G

_LN2 = 0.6931471805599453
_LN_COEFFS = (
    -0.0023178547278233695, 0.0037526242174348444, -0.004702811759193969,
    0.00813297750899996, -0.014655236243441047, 0.02636311409790927,
    -0.0493813242144068, 0.09876426659733828, -0.22222225241828053,
    0.6666666814031865, 0.4054651082139449,
)


def _ln(x):
    bits = plsc.bitcast(x, jnp.int32)
    e = lax.shift_right_logical(bits, 23) - 127
    m = plsc.bitcast((bits & 0x007FFFFF) | 0x3F800000, jnp.float32)
    w = m - 1.5
    r = jnp.full((L,), _LN_COEFFS[0], jnp.float32)
    for c in _LN_COEFFS[1:]:
        r = r * w + c
    return e.astype(jnp.float32) * _LN2 + r


def _rsqrt(s):
    r = plsc.bitcast(0x5F3759DF - lax.shift_right_logical(plsc.bitcast(s, jnp.int32), 1),
                     jnp.float32)
    for _ in range(3):
        r = r * (1.5 - 0.5 * s * r * r)
    return r


_mesh = plsc.VectorSubcoreMesh(core_axis_name="c", subcore_axis_name="s",
                               num_cores=NC, num_subcores=NS)


@functools.partial(
    pl.kernel,
    out_type=jax.ShapeDtypeStruct((NW, 4 * L), jnp.float32),
    mesh=_mesh,
    compiler_params=pltpu.CompilerParams(needs_layout_passes=False,
                                         use_tc_tiling_on_sc=False),
    scratch_types=[
        pltpu.VMEM((E_W,), jnp.int32),
        pltpu.VMEM((E_W,), jnp.int32),
        pltpu.VMEM((N_NODES,), jnp.float32),
        pltpu.VMEM((G, D_FEAT), jnp.bfloat16),
        pltpu.VMEM((G, D_FEAT), jnp.bfloat16),
        pltpu.VMEM((G, D_FEAT), jnp.bfloat16),
        pltpu.VMEM((G, D_FEAT), jnp.bfloat16),
        pltpu.VMEM((4 * L,), jnp.float32),
        pltpu.VMEM((4 * L,), jnp.float32),
        pltpu.SemaphoreType.DMA,
        pltpu.SemaphoreType.DMA,
        pltpu.SemaphoreType.DMA,
        pltpu.SemaphoreType.DMA,
    ],
)
def _partials(e0_hbm, e1_hbm, probas_hbm, feats_hbm, out_hbm,
              e0_v, e1_v, probas_v, r0a, r1a, r0b, r1b, ssqbuf_v, out_v,
              s0a, s1a, s0b, s1b):
    lanes = lax.iota(jnp.int32, L)
    wid = lax.axis_index("s") * NC + lax.axis_index("c")
    base = wid * E_W
    pltpu.sync_copy(probas_hbm, probas_v)
    pltpu.sync_copy(e0_hbm.at[pl.ds(base, E_W)], e0_v)
    pltpu.sync_copy(e1_hbm.at[pl.ds(base, E_W)], e1_v)

    def issue(g, r0, r1, s0, s1):
        off = g * G
        pltpu.async_copy(feats_hbm.at[e0_v.at[pl.ds(off, G)]], r0, s0)
        pltpu.async_copy(feats_hbm.at[e1_v.at[pl.ds(off, G)]], r1, s1)

    def wait(r0, r1, s0, s1):
        pltpu.make_async_copy(feats_hbm.at[e0_v.at[pl.ds(0, G)]], r0, s0).wait()
        pltpu.make_async_copy(feats_hbm.at[e1_v.at[pl.ds(0, G)]], r1, s1).wait()

    def compute(g, r0, r1, carry):
        off = g * G

        def sub_body(sub, carry):
            sp, sn, cp, cn = carry
            base_row = sub * L
            zero = jnp.zeros((L,), jnp.float32)
            for p in range(4):
                ssqbuf_v[pl.ds(p * L, L)] = zero
            pidx = (lanes & 3) * L
            for eb in range(0, L, 4):
                accs = [None] * 4
                for j in range(D_FEAT // (2 * L)):
                    for k in range(4):
                        row = base_row + eb + k
                        dv = (r0[row, pl.ds(j * 2 * L, 2 * L)]
                              - r1[row, pl.ds(j * 2 * L, 2 * L)])
                        sq = dv * dv
                        sq_a, sq_b = plsc.unpack(
                            sq, format=plsc.PackFormat.INTERLEAVED)
                        part = sq_a + sq_b
                        accs[k] = part if accs[k] is None else accs[k] + part
                for k in range(4):
                    plsc.addupdate_scatter(ssqbuf_v, [pidx + (eb + k)], accs[k])
            ssq = (ssqbuf_v[pl.ds(0, L)] + ssqbuf_v[pl.ds(L, L)]
                   + ssqbuf_v[pl.ds(2 * L, L)] + ssqbuf_v[pl.ds(3 * L, L)])
            i0 = e0_v[pl.ds(off + sub * L, L)]
            i1 = e1_v[pl.ds(off + sub * L, L)]
            p0 = plsc.load_gather(probas_v, [i0])
            p1 = plsc.load_gather(probas_v, [i1])
            hi0, hi1 = p0 >= HI, p1 >= HI
            lo0, lo1 = p0 < LO, p1 < LO
            sim = (hi0 & hi1) | (lo0 & lo1)
            dis = (hi0 & lo1) | (hi1 & lo0)
            norm = jnp.where(ssq > 1e-37, ssq * _rsqrt(ssq), 0.0)
            u = jnp.exp(-norm)
            t = 1.0 - u
            lnt = _ln(jnp.maximum(t, 1e-30))
            neg_t = jnp.where(t > 0.0, jnp.minimum(-lnt, 100.0), 100.0)
            sp = sp + jnp.where(sim, jnp.minimum(norm, 100.0), 0.0)
            sn = sn + jnp.where(dis, neg_t, 0.0)
            cp = cp + jnp.where(sim, 1.0, 0.0)
            cn = cn + jnp.where(dis, 1.0, 0.0)
            return sp, sn, cp, cn

        return lax.fori_loop(0, G // L, sub_body, carry)

    issue(0, r0a, r1a, s0a, s1a)
    z = jnp.zeros((L,), jnp.float32)

    def block2(k, carry):
        g0 = 2 * k
        issue(g0 + 1, r0b, r1b, s0b, s1b)
        wait(r0a, r1a, s0a, s1a)
        carry = compute(g0, r0a, r1a, carry)
        issue(g0 + 2, r0a, r1a, s0a, s1a)
        wait(r0b, r1b, s0b, s1b)
        return compute(g0 + 1, r0b, r1b, carry)

    carry = lax.fori_loop(0, NG // 2, block2, (z, z, z, z))
    wait(r0a, r1a, s0a, s1a)
    sp, sn, cp, cn = compute(NG - 1, r0a, r1a, carry)
    out_v[pl.ds(0, L)] = sp
    out_v[pl.ds(L, L)] = sn
    out_v[pl.ds(2 * L, L)] = cp
    out_v[pl.ds(3 * L, L)] = cn
    pltpu.sync_copy(out_v, out_hbm.at[wid])


def kernel(edges_nn, probas, feats):
    e0 = edges_nn[:, 0].astype(jnp.int32)
    e1 = edges_nn[:, 1].astype(jnp.int32)
    parts = _partials(e0, e1, probas, feats.astype(jnp.bfloat16))
    q = parts.reshape(NW, 4, L).sum(axis=(0, 2))
    s_pos, s_neg, n_pos, n_neg = q[0], q[1], q[2], q[3]
    n_max = jnp.maximum(n_pos, n_neg)
    pos_w = jnp.where(n_pos > 0, n_max / n_pos, 0.0)
    neg_w = jnp.where(n_neg > 0, n_max / n_neg, 0.0)
    return (pos_w * s_pos + neg_w * s_neg) / (n_pos + n_neg)


# speedup vs baseline: 2.6325x; 1.0885x over previous
import functools

import jax
import jax.numpy as jnp
from jax import lax
from jax.experimental import pallas as pl
from jax.experimental.pallas import tpu as pltpu
from jax.experimental.pallas import tpu_sc as plsc

N_NODES = 10000
N_EDGES = 320000
D_FEAT = 128
LO, HI = 0.6, 0.8

NC, NS, L = 2, 16, 16
NW = NC * NS
E_W = N_EDGES // NW
G = 80
NG = E_W // G

_LN2 = 0.6931471805599453
_LN_COEFFS = (
    -0.0023178547278233695, 0.0037526242174348444, -0.004702811759193969,
    0.00813297750899996, -0.014655236243441047, 0.02636311409790927,
    -0.0493813242144068, 0.09876426659733828, -0.22222225241828053,
    0.6666666814031865, 0.4054651082139449,
)


def _ln(x):
    bits = plsc.bitcast(x, jnp.int32)
    e = lax.shift_right_logical(bits, 23) - 127
    m = plsc.bitcast((bits & 0x007FFFFF) | 0x3F800000, jnp.float32)
    w = m - 1.5
    r = jnp.full((L,), _LN_COEFFS[0], jnp.float32)
    for c in _LN_COEFFS[1:]:
        r = r * w + c
    return e.astype(jnp.float32) * _LN2 + r


def _rsqrt(s):
    r = plsc.bitcast(0x5F3759DF - lax.shift_right_logical(plsc.bitcast(s, jnp.int32), 1),
                     jnp.float32)
    for _ in range(3):
        r = r * (1.5 - 0.5 * s * r * r)
    return r


_mesh = plsc.VectorSubcoreMesh(core_axis_name="c", subcore_axis_name="s",
                               num_cores=NC, num_subcores=NS)


@functools.partial(
    pl.kernel,
    out_type=jax.ShapeDtypeStruct((NW, 4 * L), jnp.float32),
    mesh=_mesh,
    compiler_params=pltpu.CompilerParams(needs_layout_passes=False,
                                         use_tc_tiling_on_sc=False),
    scratch_types=[
        pltpu.VMEM((E_W,), jnp.int32),
        pltpu.VMEM((E_W,), jnp.int32),
        pltpu.VMEM((N_NODES,), jnp.float32),
        pltpu.VMEM((G, D_FEAT), jnp.bfloat16),
        pltpu.VMEM((G, D_FEAT), jnp.bfloat16),
        pltpu.VMEM((G, D_FEAT), jnp.bfloat16),
        pltpu.VMEM((G, D_FEAT), jnp.bfloat16),
        pltpu.VMEM((80,), jnp.float32),
        pltpu.VMEM((4 * L,), jnp.float32),
        pltpu.SemaphoreType.DMA,
        pltpu.SemaphoreType.DMA,
        pltpu.SemaphoreType.DMA,
        pltpu.SemaphoreType.DMA,
    ],
)
def _partials(e0_hbm, e1_hbm, probas_hbm, feats_hbm, out_hbm,
              e0_v, e1_v, probas_v, r0a, r1a, r0b, r1b, ssqbuf_v, out_v,
              s0a, s1a, s0b, s1b):
    lanes = lax.iota(jnp.int32, L)
    wid = lax.axis_index("s") * NC + lax.axis_index("c")
    base = wid * E_W
    pltpu.sync_copy(probas_hbm, probas_v)
    pltpu.sync_copy(e0_hbm.at[pl.ds(base, E_W)], e0_v)
    pltpu.sync_copy(e1_hbm.at[pl.ds(base, E_W)], e1_v)

    def issue(g, r0, r1, s0, s1):
        off = g * G
        pltpu.async_copy(feats_hbm.at[e0_v.at[pl.ds(off, G)]], r0, s0)
        pltpu.async_copy(feats_hbm.at[e1_v.at[pl.ds(off, G)]], r1, s1)

    def wait(r0, r1, s0, s1):
        pltpu.make_async_copy(feats_hbm.at[e0_v.at[pl.ds(0, G)]], r0, s0).wait()
        pltpu.make_async_copy(feats_hbm.at[e1_v.at[pl.ds(0, G)]], r1, s1).wait()

    def compute(g, r0, r1, carry):
        off = g * G

        def sub_body(sub, carry):
            sp, sn, cp, cn = carry
            base_row = sub * L
            zero = jnp.zeros((L,), jnp.float32)
            for p in range(5):
                ssqbuf_v[pl.ds(p * L, L)] = zero
            pidx = (lanes & 3) * 17
            for eb in range(0, L, 4):
                accs = [None] * 4
                for j in range(D_FEAT // (2 * L)):
                    for k in range(4):
                        row = base_row + eb + k
                        dv = (r0[row, pl.ds(j * 2 * L, 2 * L)]
                              - r1[row, pl.ds(j * 2 * L, 2 * L)])
                        sq = dv * dv
                        sq_a, sq_b = plsc.unpack(
                            sq, format=plsc.PackFormat.INTERLEAVED)
                        part = sq_a + sq_b
                        accs[k] = part if accs[k] is None else accs[k] + part
                for k in range(4):
                    plsc.addupdate_scatter(ssqbuf_v, [pidx + (eb + k)], accs[k])
            ssq = (ssqbuf_v[pl.ds(0, L)] + ssqbuf_v[pl.ds(17, L)]
                   + ssqbuf_v[pl.ds(34, L)] + ssqbuf_v[pl.ds(51, L)])
            i0 = e0_v[pl.ds(off + sub * L, L)]
            i1 = e1_v[pl.ds(off + sub * L, L)]
            p0 = plsc.load_gather(probas_v, [i0])
            p1 = plsc.load_gather(probas_v, [i1])
            hi0, hi1 = p0 >= HI, p1 >= HI
            lo0, lo1 = p0 < LO, p1 < LO
            sim = (hi0 & hi1) | (lo0 & lo1)
            dis = (hi0 & lo1) | (hi1 & lo0)
            norm = jnp.where(ssq > 1e-37, ssq * _rsqrt(ssq), 0.0)
            u = jnp.exp(-norm)
            t = 1.0 - u
            lnt = _ln(jnp.maximum(t, 1e-30))
            neg_t = jnp.where(t > 0.0, jnp.minimum(-lnt, 100.0), 100.0)
            sp = sp + jnp.where(sim, jnp.minimum(norm, 100.0), 0.0)
            sn = sn + jnp.where(dis, neg_t, 0.0)
            cp = cp + jnp.where(sim, 1.0, 0.0)
            cn = cn + jnp.where(dis, 1.0, 0.0)
            return sp, sn, cp, cn

        return lax.fori_loop(0, G // L, sub_body, carry)

    issue(0, r0a, r1a, s0a, s1a)
    z = jnp.zeros((L,), jnp.float32)

    def block2(k, carry):
        g0 = 2 * k
        issue(g0 + 1, r0b, r1b, s0b, s1b)
        wait(r0a, r1a, s0a, s1a)
        carry = compute(g0, r0a, r1a, carry)
        issue(g0 + 2, r0a, r1a, s0a, s1a)
        wait(r0b, r1b, s0b, s1b)
        return compute(g0 + 1, r0b, r1b, carry)

    carry = lax.fori_loop(0, NG // 2, block2, (z, z, z, z))
    wait(r0a, r1a, s0a, s1a)
    sp, sn, cp, cn = compute(NG - 1, r0a, r1a, carry)
    out_v[pl.ds(0, L)] = sp
    out_v[pl.ds(L, L)] = sn
    out_v[pl.ds(2 * L, L)] = cp
    out_v[pl.ds(3 * L, L)] = cn
    pltpu.sync_copy(out_v, out_hbm.at[wid])


def kernel(edges_nn, probas, feats):
    e0 = edges_nn[:, 0].astype(jnp.int32)
    e1 = edges_nn[:, 1].astype(jnp.int32)
    parts = _partials(e0, e1, probas, feats.astype(jnp.bfloat16))
    q = parts.reshape(NW, 4, L).sum(axis=(0, 2))
    s_pos, s_neg, n_pos, n_neg = q[0], q[1], q[2], q[3]
    n_max = jnp.maximum(n_pos, n_neg)
    pos_w = jnp.where(n_pos > 0, n_max / n_pos, 0.0)
    neg_w = jnp.where(n_neg > 0, n_max / n_neg, 0.0)
    return (pos_w * s_pos + neg_w * s_neg) / (n_pos + n_neg)


# speedup vs baseline: 2.7818x; 1.0567x over previous
import functools

import jax
import jax.numpy as jnp
from jax import lax
from jax.experimental import pallas as pl
from jax.experimental.pallas import tpu as pltpu
from jax.experimental.pallas import tpu_sc as plsc

N_NODES = 10000
N_EDGES = 320000
D_FEAT = 128
LO, HI = 0.6, 0.8

NC, NS, L = 2, 16, 16
NW = NC * NS
E_W = N_EDGES // NW
G = 80
NG = E_W // G

_LN2 = 0.6931471805599453
_LN_COEFFS = (
    -0.0023178547278233695, 0.0037526242174348444, -0.004702811759193969,
    0.00813297750899996, -0.014655236243441047, 0.02636311409790927,
    -0.0493813242144068, 0.09876426659733828, -0.22222225241828053,
    0.6666666814031865, 0.4054651082139449,
)


def _ln(x):
    bits = plsc.bitcast(x, jnp.int32)
    e = lax.shift_right_logical(bits, 23) - 127
    m = plsc.bitcast((bits & 0x007FFFFF) | 0x3F800000, jnp.float32)
    w = m - 1.5
    r = jnp.full((L,), _LN_COEFFS[0], jnp.float32)
    for c in _LN_COEFFS[1:]:
        r = r * w + c
    return e.astype(jnp.float32) * _LN2 + r


def _rsqrt(s):
    r = plsc.bitcast(0x5F3759DF - lax.shift_right_logical(plsc.bitcast(s, jnp.int32), 1),
                     jnp.float32)
    for _ in range(3):
        r = r * (1.5 - 0.5 * s * r * r)
    return r


_mesh = plsc.VectorSubcoreMesh(core_axis_name="c", subcore_axis_name="s",
                               num_cores=NC, num_subcores=NS)


@functools.partial(
    pl.kernel,
    out_type=jax.ShapeDtypeStruct((NW, 4 * L), jnp.float32),
    mesh=_mesh,
    compiler_params=pltpu.CompilerParams(needs_layout_passes=False,
                                         use_tc_tiling_on_sc=False),
    scratch_types=[
        pltpu.VMEM((E_W,), jnp.int32),
        pltpu.VMEM((E_W,), jnp.int32),
        pltpu.VMEM((N_NODES,), jnp.float32),
        pltpu.VMEM((G, D_FEAT), jnp.bfloat16),
        pltpu.VMEM((G, D_FEAT), jnp.bfloat16),
        pltpu.VMEM((G, D_FEAT), jnp.bfloat16),
        pltpu.VMEM((G, D_FEAT), jnp.bfloat16),
        pltpu.VMEM((144,), jnp.float32),
        pltpu.VMEM((4 * L,), jnp.float32),
        pltpu.SemaphoreType.DMA,
        pltpu.SemaphoreType.DMA,
        pltpu.SemaphoreType.DMA,
        pltpu.SemaphoreType.DMA,
    ],
)
def _partials(e0_hbm, e1_hbm, probas_hbm, feats_hbm, out_hbm,
              e0_v, e1_v, probas_v, r0a, r1a, r0b, r1b, ssqbuf_v, out_v,
              s0a, s1a, s0b, s1b):
    lanes = lax.iota(jnp.int32, L)
    wid = lax.axis_index("s") * NC + lax.axis_index("c")
    base = wid * E_W
    pltpu.sync_copy(probas_hbm, probas_v)
    pltpu.sync_copy(e0_hbm.at[pl.ds(base, E_W)], e0_v)
    pltpu.sync_copy(e1_hbm.at[pl.ds(base, E_W)], e1_v)

    def issue(g, r0, r1, s0, s1):
        off = g * G
        pltpu.async_copy(feats_hbm.at[e0_v.at[pl.ds(off, G)]], r0, s0)
        pltpu.async_copy(feats_hbm.at[e1_v.at[pl.ds(off, G)]], r1, s1)

    def wait(r0, r1, s0, s1):
        pltpu.make_async_copy(feats_hbm.at[e0_v.at[pl.ds(0, G)]], r0, s0).wait()
        pltpu.make_async_copy(feats_hbm.at[e1_v.at[pl.ds(0, G)]], r1, s1).wait()

    def compute(g, r0, r1, carry):
        off = g * G

        def sub_body(sub, carry):
            sp, sn, cp, cn = carry
            base_row = sub * L
            zero = jnp.zeros((L,), jnp.float32)
            for p in range(9):
                ssqbuf_v[pl.ds(p * L, L)] = zero
            pidx = (lanes & 7) * 17
            for eb in range(0, L, 4):
                accs = [None] * 4
                for j in range(D_FEAT // (2 * L)):
                    for k in range(4):
                        row = base_row + eb + k
                        dv = (r0[row, pl.ds(j * 2 * L, 2 * L)]
                              - r1[row, pl.ds(j * 2 * L, 2 * L)])
                        sq = dv * dv
                        sq_a, sq_b = plsc.unpack(
                            sq, format=plsc.PackFormat.INTERLEAVED)
                        part = sq_a + sq_b
                        accs[k] = part if accs[k] is None else accs[k] + part
                for k in range(4):
                    plsc.addupdate_scatter(ssqbuf_v, [pidx + (eb + k)], accs[k])
            ssq = ((ssqbuf_v[pl.ds(0, L)] + ssqbuf_v[pl.ds(17, L)])
                   + (ssqbuf_v[pl.ds(34, L)] + ssqbuf_v[pl.ds(51, L)])
                   + ((ssqbuf_v[pl.ds(68, L)] + ssqbuf_v[pl.ds(85, L)])
                      + (ssqbuf_v[pl.ds(102, L)] + ssqbuf_v[pl.ds(119, L)])))
            i0 = e0_v[pl.ds(off + sub * L, L)]
            i1 = e1_v[pl.ds(off + sub * L, L)]
            p0 = plsc.load_gather(probas_v, [i0])
            p1 = plsc.load_gather(probas_v, [i1])
            hi0, hi1 = p0 >= HI, p1 >= HI
            lo0, lo1 = p0 < LO, p1 < LO
            sim = (hi0 & hi1) | (lo0 & lo1)
            dis = (hi0 & lo1) | (hi1 & lo0)
            norm = jnp.where(ssq > 1e-37, ssq * _rsqrt(ssq), 0.0)
            u = jnp.exp(-norm)
            t = 1.0 - u
            lnt = _ln(jnp.maximum(t, 1e-30))
            neg_t = jnp.where(t > 0.0, jnp.minimum(-lnt, 100.0), 100.0)
            sp = sp + jnp.where(sim, jnp.minimum(norm, 100.0), 0.0)
            sn = sn + jnp.where(dis, neg_t, 0.0)
            cp = cp + jnp.where(sim, 1.0, 0.0)
            cn = cn + jnp.where(dis, 1.0, 0.0)
            return sp, sn, cp, cn

        return lax.fori_loop(0, G // L, sub_body, carry)

    issue(0, r0a, r1a, s0a, s1a)
    z = jnp.zeros((L,), jnp.float32)

    def block2(k, carry):
        g0 = 2 * k
        issue(g0 + 1, r0b, r1b, s0b, s1b)
        wait(r0a, r1a, s0a, s1a)
        carry = compute(g0, r0a, r1a, carry)
        issue(g0 + 2, r0a, r1a, s0a, s1a)
        wait(r0b, r1b, s0b, s1b)
        return compute(g0 + 1, r0b, r1b, carry)

    carry = lax.fori_loop(0, NG // 2, block2, (z, z, z, z))
    wait(r0a, r1a, s0a, s1a)
    sp, sn, cp, cn = compute(NG - 1, r0a, r1a, carry)
    out_v[pl.ds(0, L)] = sp
    out_v[pl.ds(L, L)] = sn
    out_v[pl.ds(2 * L, L)] = cp
    out_v[pl.ds(3 * L, L)] = cn
    pltpu.sync_copy(out_v, out_hbm.at[wid])


def kernel(edges_nn, probas, feats):
    e0 = edges_nn[:, 0].astype(jnp.int32)
    e1 = edges_nn[:, 1].astype(jnp.int32)
    parts = _partials(e0, e1, probas, feats.astype(jnp.bfloat16))
    q = parts.reshape(NW, 4, L).sum(axis=(0, 2))
    s_pos, s_neg, n_pos, n_neg = q[0], q[1], q[2], q[3]
    n_max = jnp.maximum(n_pos, n_neg)
    pos_w = jnp.where(n_pos > 0, n_max / n_pos, 0.0)
    neg_w = jnp.where(n_neg > 0, n_max / n_neg, 0.0)
    return (pos_w * s_pos + neg_w * s_neg) / (n_pos + n_neg)
